# Initial kernel scaffold; baseline (speedup 1.0000x reference)
import jax
import jax.numpy as jnp
from jax.experimental import pallas as pl


def kernel(indices, table):
    raise NotImplementedError("write your pallas kernel here")


# speedup vs baseline: 1.2661x; 1.2661x over previous
import functools

import jax
import jax.numpy as jnp
from jax import lax
from jax.experimental import pallas as pl
from jax.experimental.pallas import tpu as pltpu
from jax.experimental.pallas import tpu_sc as plsc


@functools.lru_cache(maxsize=None)
def _build_gather(n_rows: int, vocab: int, dim: int):
    info = plsc.get_sparse_core_info()
    nc, ns = info.num_cores, info.num_subcores
    nw = nc * ns
    chunk = 80
    per_w = n_rows // nw
    nch = per_w // chunk
    assert per_w % chunk == 0 and n_rows % nw == 0

    mesh = plsc.VectorSubcoreMesh(core_axis_name="c", subcore_axis_name="s")

    @functools.partial(
        pl.kernel,
        mesh=mesh,
        out_type=jax.ShapeDtypeStruct((n_rows, dim), jnp.float32),
        scratch_types=[
            pltpu.VMEM((nch, chunk), jnp.int32),
            pltpu.VMEM((2, chunk, dim), jnp.float32),
            pltpu.SemaphoreType.DMA,
            pltpu.SemaphoreType.DMA,
            pltpu.SemaphoreType.DMA,
            pltpu.SemaphoreType.DMA,
        ],
    )
    def gather_kernel(idx_hbm, table_hbm, out_hbm, idx_v, rows_v, g0, g1, o0, o1):
        wid = lax.axis_index("s") * nc + lax.axis_index("c")
        base = wid * per_w
        gsems = (g0, g1)
        osems = (o0, o1)
        pltpu.sync_copy(idx_hbm.at[wid], idx_v)
        gathers = {}
        outs = {}
        gathers[0] = pltpu.async_copy(
            table_hbm.at[idx_v.at[0]], rows_v.at[0], gsems[0])
        for j in range(nch):
            b = j % 2
            if j + 1 < nch:
                nb = (j + 1) % 2
                if j >= 1:
                    outs[j - 1].wait()
                gathers[j + 1] = pltpu.async_copy(
                    table_hbm.at[idx_v.at[j + 1]], rows_v.at[nb], gsems[nb])
            gathers[j].wait()
            outs[j] = pltpu.async_copy(
                rows_v.at[b], out_hbm.at[pl.ds(base + j * chunk, chunk)],
                osems[b])
        outs[nch - 2].wait()
        outs[nch - 1].wait()

    return gather_kernel


def kernel(indices, table):
    batch, seq = indices.shape
    vocab, dim = table.shape
    n_rows = batch * seq
    gather = _build_gather(n_rows, vocab, dim)
    info = plsc.get_sparse_core_info()
    nw = info.num_cores * info.num_subcores
    chunk = 80
    idx3 = indices.reshape(nw, (n_rows // nw) // chunk, chunk).astype(jnp.int32)
    rows = gather(idx3, table)
    return rows.reshape(batch, seq, dim)


# speedup vs baseline: 1.2726x; 1.0051x over previous
import functools

import jax
import jax.numpy as jnp
from jax import lax
from jax.experimental import pallas as pl
from jax.experimental.pallas import tpu as pltpu
from jax.experimental.pallas import tpu_sc as plsc


@functools.lru_cache(maxsize=None)
def _build_gather(n_rows: int, vocab: int, dim: int):
    info = plsc.get_sparse_core_info()
    nc, ns = info.num_cores, info.num_subcores
    nw = nc * ns
    chunk = 80
    nbuf = 3
    per_w = n_rows // nw
    nch = per_w // chunk
    assert per_w % chunk == 0 and n_rows % nw == 0

    mesh = plsc.VectorSubcoreMesh(core_axis_name="c", subcore_axis_name="s")

    @functools.partial(
        pl.kernel,
        mesh=mesh,
        out_type=jax.ShapeDtypeStruct((n_rows, dim), jnp.float32),
        scratch_types=[
            pltpu.VMEM((nch, chunk), jnp.int32),
            pltpu.VMEM((nbuf, chunk, dim), jnp.float32),
        ] + [pltpu.SemaphoreType.DMA] * (2 * nbuf),
    )
    def gather_kernel(idx_hbm, table_hbm, out_hbm, idx_v, rows_v, *sems):
        wid = lax.axis_index("s") * nc + lax.axis_index("c")
        base = wid * per_w
        gsems, osems = sems[:nbuf], sems[nbuf:]
        pltpu.sync_copy(idx_hbm.at[wid], idx_v)
        gathers = {}
        outs = {}
        for j in range(min(nbuf - 1, nch)):
            gathers[j] = pltpu.async_copy(
                table_hbm.at[idx_v.at[j]], rows_v.at[j], gsems[j])
        for j in range(nch):
            b = j % nbuf
            jn = j + nbuf - 1
            if jn < nch:
                bn = jn % nbuf
                if jn - nbuf >= 0:
                    outs[jn - nbuf].wait()
                gathers[jn] = pltpu.async_copy(
                    table_hbm.at[idx_v.at[jn]], rows_v.at[bn], gsems[bn])
            gathers[j].wait()
            outs[j] = pltpu.async_copy(
                rows_v.at[b], out_hbm.at[pl.ds(base + j * chunk, chunk)],
                osems[b])
        for j in range(max(0, nch - nbuf), nch):
            outs[j].wait()

    return gather_kernel


def kernel(indices, table):
    batch, seq = indices.shape
    vocab, dim = table.shape
    n_rows = batch * seq
    gather = _build_gather(n_rows, vocab, dim)
    info = plsc.get_sparse_core_info()
    nw = info.num_cores * info.num_subcores
    chunk = 80
    idx3 = indices.reshape(nw, (n_rows // nw) // chunk, chunk).astype(jnp.int32)
    rows = gather(idx3, table)
    return rows.reshape(batch, seq, dim)


# speedup vs baseline: 1.7881x; 1.4051x over previous
import functools

import jax
import jax.numpy as jnp
from jax import lax
from jax.experimental import pallas as pl
from jax.experimental.pallas import tpu as pltpu
from jax.experimental.pallas import tpu_sc as plsc


@functools.lru_cache(maxsize=None)
def _build_gather(batch: int, seq: int, vocab: int, dim: int):
    info = plsc.get_sparse_core_info()
    nc, ns = info.num_cores, info.num_subcores
    nw = nc * ns
    nbuf = 3
    per_w = batch // nw
    assert batch % nw == 0

    mesh = plsc.VectorSubcoreMesh(core_axis_name="c", subcore_axis_name="s")

    @functools.partial(
        pl.kernel,
        mesh=mesh,
        out_type=jax.ShapeDtypeStruct((batch, seq, dim), jnp.float32),
        scratch_types=[pltpu.VMEM((per_w, seq), jnp.int32)]
        + [pltpu.VMEM((seq, dim), jnp.float32)] * nbuf
        + [pltpu.SemaphoreType.DMA] * (2 * nbuf),
    )
    def gather_kernel(idx_hbm, table_hbm, out_hbm, idx_v, *bufs_and_sems):
        rows = bufs_and_sems[:nbuf]
        gsems = bufs_and_sems[nbuf:2 * nbuf]
        osems = bufs_and_sems[2 * nbuf:]
        wid = lax.axis_index("s") * nc + lax.axis_index("c")
        ebase = wid * per_w
        pltpu.sync_copy(idx_hbm.at[wid], idx_v)
        gathers = {}
        outs = {}
        for j in range(nbuf - 1):
            gathers[j] = pltpu.async_copy(
                table_hbm.at[idx_v.at[j]], rows[j], gsems[j])
        for j in range(per_w):
            b = j % nbuf
            jn = j + nbuf - 1
            if jn < per_w:
                bn = jn % nbuf
                if jn - nbuf >= 0:
                    outs[jn - nbuf].wait()
                gathers[jn] = pltpu.async_copy(
                    table_hbm.at[idx_v.at[jn]], rows[bn], gsems[bn])
            gathers[j].wait()
            outs[j] = pltpu.async_copy(
                rows[b], out_hbm.at[ebase + j], osems[b])
        for j in range(per_w - nbuf, per_w):
            outs[j].wait()

    return gather_kernel


def kernel(indices, table):
    batch, seq = indices.shape
    vocab, dim = table.shape
    gather = _build_gather(batch, seq, vocab, dim)
    info = plsc.get_sparse_core_info()
    nw = info.num_cores * info.num_subcores
    idx3 = indices.reshape(nw, batch // nw, seq).astype(jnp.int32)
    return gather(idx3, table)


# speedup vs baseline: 1.8654x; 1.0432x over previous
import functools

import jax
import jax.numpy as jnp
from jax import lax
from jax.experimental import pallas as pl
from jax.experimental.pallas import tpu as pltpu
from jax.experimental.pallas import tpu_sc as plsc


@functools.lru_cache(maxsize=None)
def _build_gather(batch: int, sp: int, vocab: int, dim: int):
    info = plsc.get_sparse_core_info()
    nc, ns = info.num_cores, info.num_subcores
    nw = nc * ns
    nbuf = 3
    per_w = batch // nw
    assert batch % nw == 0 and sp % 8 == 0 and sp <= 128

    mesh = plsc.VectorSubcoreMesh(core_axis_name="c", subcore_axis_name="s")

    @functools.partial(
        pl.kernel,
        mesh=mesh,
        out_type=jax.ShapeDtypeStruct((batch, sp, dim), jnp.float32),
        scratch_types=[pltpu.VMEM((per_w, sp), jnp.int32)]
        + [pltpu.VMEM((sp, dim), jnp.float32)] * nbuf
        + [pltpu.SemaphoreType.DMA] * (2 * nbuf),
    )
    def gather_kernel(idx_hbm, table_hbm, out_hbm, idx_v, *bufs_and_sems):
        rows = bufs_and_sems[:nbuf]
        gsems = bufs_and_sems[nbuf:2 * nbuf]
        osems = bufs_and_sems[2 * nbuf:]
        wid = lax.axis_index("s") * nc + lax.axis_index("c")
        ebase = wid * per_w
        pltpu.sync_copy(idx_hbm.at[wid], idx_v)
        gathers = {}
        outs = {}
        for j in range(nbuf - 1):
            gathers[j] = pltpu.async_copy(
                table_hbm.at[idx_v.at[j]], rows[j], gsems[j])
        for j in range(per_w):
            b = j % nbuf
            jn = j + nbuf - 1
            if jn < per_w:
                bn = jn % nbuf
                if jn - nbuf >= 0:
                    outs[jn - nbuf].wait()
                gathers[jn] = pltpu.async_copy(
                    table_hbm.at[idx_v.at[jn]], rows[bn], gsems[bn])
            gathers[j].wait()
            outs[j] = pltpu.async_copy(
                rows[b], out_hbm.at[ebase + j], osems[b])
        for j in range(per_w - nbuf, per_w):
            outs[j].wait()

    return gather_kernel


def kernel(indices, table):
    batch, seq = indices.shape
    vocab, dim = table.shape
    sp = (seq + 7) // 8 * 8
    gather = _build_gather(batch, sp, vocab, dim)
    info = plsc.get_sparse_core_info()
    nw = info.num_cores * info.num_subcores
    idx3 = indices.reshape(nw, batch // nw, seq).astype(jnp.int32)
    idx3 = jnp.pad(idx3, ((0, 0), (0, 0), (0, sp - seq)), mode="wrap")
    out = gather(idx3, table)
    return out[:, :seq, :]


# speedup vs baseline: 1.8780x; 1.0068x over previous
import functools

import jax
import jax.numpy as jnp
from jax import lax
from jax.experimental import pallas as pl
from jax.experimental.pallas import tpu as pltpu
from jax.experimental.pallas import tpu_sc as plsc


@functools.lru_cache(maxsize=None)
def _build_gather(batch: int, seq: int, vocab: int, dim: int):
    info = plsc.get_sparse_core_info()
    nc, ns = info.num_cores, info.num_subcores
    nw = nc * ns
    nbuf = 3
    per_w = batch // nw
    sm = seq - seq % 8
    tr = seq - sm
    tw = per_w * tr
    assert batch % nw == 0 and sm > 0 and tw % 8 == 0

    mesh = plsc.VectorSubcoreMesh(core_axis_name="c", subcore_axis_name="s")

    @functools.partial(
        pl.kernel,
        mesh=mesh,
        out_type=(
            jax.ShapeDtypeStruct((batch, seq, dim), jnp.float32),
            jax.ShapeDtypeStruct((batch * tr, dim), jnp.float32),
        ),
        scratch_types=[
            pltpu.VMEM((per_w, sm), jnp.int32),
            pltpu.VMEM((1, tw), jnp.int32),
            pltpu.VMEM((tw, dim), jnp.float32),
        ]
        + [pltpu.VMEM((sm, dim), jnp.float32)] * nbuf
        + [pltpu.SemaphoreType.DMA] * (2 * nbuf + 2),
    )
    def gather_kernel(idx_hbm, tidx_hbm, table_hbm, out_hbm, tail_hbm,
                      idx_v, tidx_v, tbuf, *bufs_and_sems):
        rows = bufs_and_sems[:nbuf]
        gsems = bufs_and_sems[nbuf:2 * nbuf]
        osems = bufs_and_sems[2 * nbuf:3 * nbuf]
        tsem_g, tsem_o = bufs_and_sems[3 * nbuf:]
        wid = lax.axis_index("s") * nc + lax.axis_index("c")
        ebase = wid * per_w
        pltpu.sync_copy(idx_hbm.at[wid], idx_v)
        pltpu.sync_copy(tidx_hbm.at[wid], tidx_v)
        tail_g = pltpu.async_copy(table_hbm.at[tidx_v.at[0]], tbuf, tsem_g)
        gathers = {}
        outs = {}
        for j in range(nbuf - 1):
            gathers[j] = pltpu.async_copy(
                table_hbm.at[idx_v.at[j]], rows[j], gsems[j])
        for j in range(per_w):
            b = j % nbuf
            jn = j + nbuf - 1
            if jn < per_w:
                bn = jn % nbuf
                if jn - nbuf >= 0:
                    outs[jn - nbuf].wait()
                gathers[jn] = pltpu.async_copy(
                    table_hbm.at[idx_v.at[jn]], rows[bn], gsems[bn])
            gathers[j].wait()
            outs[j] = pltpu.async_copy(
                rows[b], out_hbm.at[ebase + j, pl.ds(0, sm)], osems[b])
            if j == per_w // 2:
                tail_g.wait()
                tail_o = pltpu.async_copy(
                    tbuf, tail_hbm.at[pl.ds(wid * tw, tw)], tsem_o)
        for j in range(per_w - nbuf, per_w):
            outs[j].wait()
        tail_o.wait()

    return gather_kernel


def kernel(indices, table):
    batch, seq = indices.shape
    vocab, dim = table.shape
    gather = _build_gather(batch, seq, vocab, dim)
    info = plsc.get_sparse_core_info()
    nw = info.num_cores * info.num_subcores
    per_w = batch // nw
    sm = seq - seq % 8
    tr = seq - sm
    idx3 = indices.reshape(nw, per_w, seq).astype(jnp.int32)
    idx_main = idx3[:, :, :sm]
    idx_tail = idx3[:, :, sm:].reshape(nw, 1, per_w * tr)
    out, tail = gather(idx_main, idx_tail, table)
    tail3 = tail.reshape(batch, tr, dim)
    return lax.dynamic_update_slice(out, tail3, (0, sm, 0))
